# Initial kernel scaffold; baseline (speedup 1.0000x reference)
import jax
import jax.numpy as jnp
from jax.experimental import pallas as pl


def kernel(x, edge_index, W1, b1, W2, b2, W3, b3, W4, b4, Wc, bc):
    raise NotImplementedError("write your pallas kernel here")


# speedup vs baseline: 9.6459x; 9.6459x over previous
import functools

import jax
import jax.numpy as jnp
from jax import lax
from jax.experimental import pallas as pl
from jax.experimental.pallas import tpu as pltpu
from jax.experimental.pallas import tpu_sc as plsc

_NC = 2
_NS = 16
_CH = 80


@functools.lru_cache(maxsize=None)
def _make_agg(n_nodes: int, n_edges: int, width: int):
    nw = _NC * _NS
    epw = n_edges // nw
    assert epw * nw == n_edges and epw % _CH == 0
    nit = epw // _CH
    npad = -(-n_nodes // (8 * _NS)) * (8 * _NS)
    rpt = npad // _NS

    mesh = plsc.VectorSubcoreMesh(
        core_axis_name="c", subcore_axis_name="s",
        num_cores=_NC, num_subcores=_NS)

    def body(tab_hbm, src_hbm, dst_hbm, zeros_hbm, out_hbm,
             src_v, dst_v, rows_v, acc_sh, sem):
        c = lax.axis_index("c")
        s = lax.axis_index("s")
        wid = s * _NC + c
        pltpu.sync_copy(zeros_hbm.at[pl.ds(s * rpt, rpt)],
                        acc_sh.at[pl.ds(s * rpt, rpt)])
        plsc.subcore_barrier()

        def step(i, carry):
            base = wid * epw + i * _CH
            pltpu.sync_copy(src_hbm.at[pl.ds(base, _CH)], src_v)
            pltpu.sync_copy(dst_hbm.at[pl.ds(base, _CH)], dst_v)
            pltpu.async_copy(tab_hbm.at[src_v], rows_v, sem).wait()
            pltpu.sync_copy(rows_v, acc_sh.at[dst_v], add=True)
            return carry

        lax.fori_loop(0, nit, step, 0)
        plsc.subcore_barrier()
        pltpu.sync_copy(acc_sh.at[pl.ds(s * rpt, rpt)],
                        out_hbm.at[c, pl.ds(s * rpt, rpt)])

    return pl.kernel(
        body,
        out_type=jax.ShapeDtypeStruct((_NC, npad, width), jnp.float32),
        mesh=mesh,
        scratch_types=[
            pltpu.VMEM((_CH,), jnp.int32),
            pltpu.VMEM((_CH,), jnp.int32),
            pltpu.VMEM((_CH, width), jnp.float32),
            pltpu.VMEM_SHARED((npad, width), jnp.float32),
            pltpu.SemaphoreType.DMA,
        ],
        compiler_params=pltpu.CompilerParams(use_tc_tiling_on_sc=False),
    )


_BR = 1000


def _full(shape):
    return pl.BlockSpec(shape, lambda i: (0,) * len(shape))


def _rows(shape3=None, width=128):
    if shape3:
        return pl.BlockSpec((shape3[0], _BR, shape3[2]), lambda i: (0, i, 0))
    return pl.BlockSpec((_BR, width), lambda i: (i, 0))


def _ka_body(x_ref, w_ref, dc_ref, hp_ref, dis_ref):
    deg = 1.0 + dc_ref[0, :, 0:1] + dc_ref[1, :, 0:1]
    dis = lax.rsqrt(deg)
    h = jnp.dot(x_ref[...], w_ref[...], preferred_element_type=jnp.float32)
    hp_ref[...] = h * dis
    dis_ref[...] = dis


def _kb_body(act, pad_out, agg_ref, hp_ref, dis_ref, b_ref, w_ref, out_ref):
    dis = dis_ref[...]
    t = dis * (agg_ref[0] + agg_ref[1] + hp_ref[...]) + b_ref[...]
    t = act(t)
    v = dis * jnp.dot(t, w_ref[...], preferred_element_type=jnp.float32)
    if pad_out:
        v = jnp.concatenate([v, jnp.zeros_like(v)], axis=1)
    out_ref[...] = v


def _kc_body(agg_ref, hp_ref, dis_ref, b_ref, wc_ref, bc_ref, out_ref):
    t = (agg_ref[0] + agg_ref[1] + hp_ref[...])[:, :8]
    t = dis_ref[...] * t + b_ref[...]
    t = jnp.where(t >= 0, t, 0.01 * t)
    o = jnp.dot(t, wc_ref[...], preferred_element_type=jnp.float32) + bc_ref[...]
    out_ref[...] = jnp.where(o > 0, o, jnp.exp(o) - 1.0)


def _act_id(t):
    return t


def _act_tanh(t):
    return jnp.tanh(t)


def _act_leaky(t):
    return jnp.where(t >= 0, t, 0.01 * t)


def kernel(x, edge_index, W1, b1, W2, b2, W3, b3, W4, b4, Wc, bc):
    n, d = x.shape
    e = edge_index.shape[1]
    grid = (n // _BR,)
    src = edge_index[0].astype(jnp.int32)
    dst = edge_index[1].astype(jnp.int32)
    npad = -(-n // (8 * _NS)) * (8 * _NS)
    ones16 = jnp.ones((n, 16), jnp.float32)
    zeros16 = jnp.zeros((npad, 16), jnp.float32)
    zerosd = jnp.zeros((npad, d), jnp.float32)

    agg16 = _make_agg(n, e, 16)
    aggd = _make_agg(n, e, d)

    dc = agg16(ones16, src, dst, zeros16)

    hp1, dis = pl.pallas_call(
        _ka_body,
        grid=grid,
        in_specs=[_rows(width=d), _full((d, d)), _rows((_NC, n, 16))],
        out_specs=[_rows(width=d), _rows(width=1)],
        out_shape=[jax.ShapeDtypeStruct((n, d), jnp.float32),
                   jax.ShapeDtypeStruct((n, 1), jnp.float32)],
    )(x, W1, dc)

    def mid(aggp, hp, b, w, act, dout, pad_out):
        wout = 2 * dout if pad_out else dout
        return pl.pallas_call(
            functools.partial(_kb_body, act, pad_out),
            grid=grid,
            in_specs=[_rows((_NC, n, d)), _rows(width=d), _rows(width=1),
                      _full((1, d)), _full((d, dout))],
            out_specs=_rows(width=wout),
            out_shape=jax.ShapeDtypeStruct((n, wout), jnp.float32),
        )(aggp, hp, dis, b.reshape(1, d), w)

    a1 = aggd(hp1, src, dst, zerosd)
    hp2 = mid(a1, hp1, b1, W2, _act_id, d, False)
    a2 = aggd(hp2, src, dst, zerosd)
    hp3 = mid(a2, hp2, b2, W3, _act_tanh, d, False)
    a3 = aggd(hp3, src, dst, zerosd)
    hp4 = mid(a3, hp3, b3, W4, _act_leaky, 8, True)
    a4 = agg16(hp4, src, dst, zeros16)

    out = pl.pallas_call(
        _kc_body,
        grid=grid,
        in_specs=[_rows((_NC, n, 16)), _rows(width=16), _rows(width=1),
                  _full((1, 8)), _full((8, 1)), _full((1, 1))],
        out_specs=_rows(width=1),
        out_shape=jax.ShapeDtypeStruct((n, 1), jnp.float32),
    )(a4, hp4, dis, b4.reshape(1, 8), Wc, bc.reshape(1, 1))
    return out


# speedup vs baseline: 25.7989x; 2.6746x over previous
import functools

import jax
import jax.numpy as jnp
from jax import lax
from jax.experimental import pallas as pl
from jax.experimental.pallas import tpu as pltpu
from jax.experimental.pallas import tpu_sc as plsc

_NC = 2
_NS = 16
_CH = 80


@functools.lru_cache(maxsize=None)
def _make_agg(n_nodes: int, n_edges: int, width: int, gather: bool):
    nw = _NC * _NS
    epw = n_edges // nw
    assert epw * nw == n_edges and epw % _CH == 0
    nit = epw // _CH
    cpt = nit
    assert nit % 2 == 1
    npad = -(-n_nodes // (8 * _NS)) * (8 * _NS)
    rpt = npad // _NS

    mesh = plsc.VectorSubcoreMesh(
        core_axis_name="c", subcore_axis_name="s",
        num_cores=_NC, num_subcores=_NS)

    def body(tab_hbm, src_hbm, dst_hbm, zeros_hbm, out_hbm,
             src_v, dst_v, rows_a, rows_b, acc_sh, sem_a, sem_b, sem_i):
        c = lax.axis_index("c")
        s = lax.axis_index("s")
        wid = s * _NC + c
        idx = pltpu.async_copy(dst_hbm.at[pl.ds(wid * cpt, cpt)], dst_v, sem_i)
        if gather:
            idx2 = pltpu.async_copy(
                src_hbm.at[pl.ds(wid * cpt, cpt)], src_v, sem_i)
        else:
            idx2 = None
            pltpu.sync_copy(tab_hbm, rows_a)
        pltpu.sync_copy(zeros_hbm.at[pl.ds(s * rpt, rpt)],
                        acc_sh.at[pl.ds(s * rpt, rpt)])
        idx.wait()
        if idx2 is not None:
            idx2.wait()
        plsc.subcore_barrier()

        if gather:
            def g_issue(i, buf, sem):
                pltpu.async_copy(tab_hbm.at[src_v.at[i]], buf, sem)

            def g_wait(i, buf, sem):
                pltpu.make_async_copy(tab_hbm.at[src_v.at[i]], buf, sem).wait()

            g_issue(0, rows_a, sem_a)

            def pair(j, carry):
                i = 2 * j
                g_issue(i + 1, rows_b, sem_b)
                g_wait(i, rows_a, sem_a)
                pltpu.sync_copy(rows_a, acc_sh.at[dst_v.at[i]], add=True)
                g_issue(i + 2, rows_a, sem_a)
                g_wait(i + 1, rows_b, sem_b)
                pltpu.sync_copy(rows_b, acc_sh.at[dst_v.at[i + 1]], add=True)
                return carry

            lax.fori_loop(0, (nit - 1) // 2, pair, 0)
            g_wait(nit - 1, rows_a, sem_a)
            pltpu.sync_copy(rows_a, acc_sh.at[dst_v.at[nit - 1]], add=True)
        else:
            def step(i, carry):
                pltpu.sync_copy(rows_a, acc_sh.at[dst_v.at[i]], add=True)
                return carry

            lax.fori_loop(0, nit, step, 0)

        plsc.subcore_barrier()
        pltpu.sync_copy(acc_sh.at[pl.ds(s * rpt, rpt)],
                        out_hbm.at[c, pl.ds(s * rpt, rpt)])

    return pl.kernel(
        body,
        out_type=jax.ShapeDtypeStruct((_NC, npad, width), jnp.float32),
        mesh=mesh,
        scratch_types=[
            pltpu.VMEM((cpt, _CH), jnp.int32),
            pltpu.VMEM((cpt, _CH), jnp.int32),
            pltpu.VMEM((_CH, width), jnp.float32),
            pltpu.VMEM((_CH, width), jnp.float32),
            pltpu.VMEM_SHARED((npad, width), jnp.float32),
            pltpu.SemaphoreType.DMA,
            pltpu.SemaphoreType.DMA,
            pltpu.SemaphoreType.DMA,
        ],
        compiler_params=pltpu.CompilerParams(use_tc_tiling_on_sc=False),
    )


_BR = 1000


def _full(shape):
    return pl.BlockSpec(shape, lambda i: (0,) * len(shape))


def _rows(shape3=None, width=128):
    if shape3:
        return pl.BlockSpec((shape3[0], _BR, shape3[2]), lambda i: (0, i, 0))
    return pl.BlockSpec((_BR, width), lambda i: (i, 0))


def _ka_body(x_ref, w_ref, dc_ref, hp_ref, dis_ref):
    deg = 1.0 + dc_ref[0, :, 0:1] + dc_ref[1, :, 0:1]
    dis = lax.rsqrt(deg)
    h = jnp.dot(x_ref[...], w_ref[...], preferred_element_type=jnp.float32)
    hp_ref[...] = h * dis
    dis_ref[...] = dis


def _kb_body(act, pad_out, agg_ref, hp_ref, dis_ref, b_ref, w_ref, out_ref):
    dis = dis_ref[...]
    t = dis * (agg_ref[0] + agg_ref[1] + hp_ref[...]) + b_ref[...]
    t = act(t)
    v = dis * jnp.dot(t, w_ref[...], preferred_element_type=jnp.float32)
    if pad_out:
        v = jnp.concatenate([v, jnp.zeros_like(v)], axis=1)
    out_ref[...] = v


def _kc_body(agg_ref, hp_ref, dis_ref, b_ref, wc_ref, bc_ref, out_ref):
    t = (agg_ref[0] + agg_ref[1] + hp_ref[...])[:, :8]
    t = dis_ref[...] * t + b_ref[...]
    t = jnp.where(t >= 0, t, 0.01 * t)
    o = jnp.dot(t, wc_ref[...], preferred_element_type=jnp.float32) + bc_ref[...]
    out_ref[...] = jnp.where(o > 0, o, jnp.exp(o) - 1.0)


def _act_id(t):
    return t


def _act_tanh(t):
    return jnp.tanh(t)


def _act_leaky(t):
    return jnp.where(t >= 0, t, 0.01 * t)


def kernel(x, edge_index, W1, b1, W2, b2, W3, b3, W4, b4, Wc, bc):
    n, d = x.shape
    e = edge_index.shape[1]
    grid = (n // _BR,)
    src = edge_index[0].astype(jnp.int32)
    dst = edge_index[1].astype(jnp.int32)
    npad = -(-n // (8 * _NS)) * (8 * _NS)
    src2 = src.reshape(-1, _CH)
    dst2 = dst.reshape(-1, _CH)
    ones_blk = jnp.ones((_CH, 16), jnp.float32)
    zeros16 = jnp.zeros((npad, 16), jnp.float32)
    zerosd = jnp.zeros((npad, d), jnp.float32)

    deg16 = _make_agg(n, e, 16, False)
    agg16 = _make_agg(n, e, 16, True)
    aggd = _make_agg(n, e, d, True)

    dc = deg16(ones_blk, src2, dst2, zeros16)

    hp1, dis = pl.pallas_call(
        _ka_body,
        grid=grid,
        in_specs=[_rows(width=d), _full((d, d)), _rows((_NC, n, 16))],
        out_specs=[_rows(width=d), _rows(width=1)],
        out_shape=[jax.ShapeDtypeStruct((n, d), jnp.float32),
                   jax.ShapeDtypeStruct((n, 1), jnp.float32)],
    )(x, W1, dc)

    def mid(aggp, hp, b, w, act, dout, pad_out):
        wout = 2 * dout if pad_out else dout
        return pl.pallas_call(
            functools.partial(_kb_body, act, pad_out),
            grid=grid,
            in_specs=[_rows((_NC, n, d)), _rows(width=d), _rows(width=1),
                      _full((1, d)), _full((d, dout))],
            out_specs=_rows(width=wout),
            out_shape=jax.ShapeDtypeStruct((n, wout), jnp.float32),
        )(aggp, hp, dis, b.reshape(1, d), w)

    a1 = aggd(hp1, src2, dst2, zerosd)
    hp2 = mid(a1, hp1, b1, W2, _act_id, d, False)
    a2 = aggd(hp2, src2, dst2, zerosd)
    hp3 = mid(a2, hp2, b2, W3, _act_tanh, d, False)
    a3 = aggd(hp3, src2, dst2, zerosd)
    hp4 = mid(a3, hp3, b3, W4, _act_leaky, 8, True)
    a4 = agg16(hp4, src2, dst2, zeros16)

    out = pl.pallas_call(
        _kc_body,
        grid=grid,
        in_specs=[_rows((_NC, n, 16)), _rows(width=16), _rows(width=1),
                  _full((1, 8)), _full((8, 1)), _full((1, 1))],
        out_specs=_rows(width=1),
        out_shape=jax.ShapeDtypeStruct((n, 1), jnp.float32),
    )(a4, hp4, dis, b4.reshape(1, 8), Wc, bc.reshape(1, 1))
    return out


# speedup vs baseline: 28.2181x; 1.0938x over previous
import functools

import jax
import jax.numpy as jnp
from jax import lax
from jax.experimental import pallas as pl
from jax.experimental.pallas import tpu as pltpu
from jax.experimental.pallas import tpu_sc as plsc

_NC = 2
_NS = 16
_CHW = 80
_CHN = 400


@functools.lru_cache(maxsize=None)
def _make_agg(n_nodes: int, n_edges: int, width: int, gather: bool, ch: int):
    nw = _NC * _NS
    epw = n_edges // nw
    assert epw * nw == n_edges and epw % ch == 0
    nit = epw // ch
    cpt = nit
    assert nit % 2 == 1
    npad = -(-n_nodes // (8 * _NS)) * (8 * _NS)
    rpt = npad // _NS

    mesh = plsc.VectorSubcoreMesh(
        core_axis_name="c", subcore_axis_name="s",
        num_cores=_NC, num_subcores=_NS)

    def body(ei_hbm, tab_hbm, zeros_hbm, out_hbm,
             src_v, dst_v, rows_a, rows_b, acc_sh, sem_a, sem_b, sem_i):
        c = lax.axis_index("c")
        s = lax.axis_index("s")
        wid = s * _NC + c
        idx = pltpu.async_copy(ei_hbm.at[1, wid], dst_v, sem_i)
        if gather:
            idx2 = pltpu.async_copy(ei_hbm.at[0, wid], src_v, sem_i)
        else:
            idx2 = None
            pltpu.sync_copy(tab_hbm, rows_a)
        pltpu.sync_copy(zeros_hbm.at[pl.ds(s * rpt, rpt)],
                        acc_sh.at[pl.ds(s * rpt, rpt)])
        idx.wait()
        if idx2 is not None:
            idx2.wait()
        plsc.subcore_barrier()

        if gather:
            def g_issue(i, buf, sem):
                pltpu.async_copy(tab_hbm.at[src_v.at[i]], buf, sem)

            def g_wait(i, buf, sem):
                pltpu.make_async_copy(tab_hbm.at[src_v.at[i]], buf, sem).wait()

            g_issue(0, rows_a, sem_a)

            def pair(j, carry):
                i = 2 * j
                g_issue(i + 1, rows_b, sem_b)
                g_wait(i, rows_a, sem_a)
                pltpu.sync_copy(rows_a, acc_sh.at[dst_v.at[i]], add=True)
                g_issue(i + 2, rows_a, sem_a)
                g_wait(i + 1, rows_b, sem_b)
                pltpu.sync_copy(rows_b, acc_sh.at[dst_v.at[i + 1]], add=True)
                return carry

            lax.fori_loop(0, (nit - 1) // 2, pair, 0)
            g_wait(nit - 1, rows_a, sem_a)
            pltpu.sync_copy(rows_a, acc_sh.at[dst_v.at[nit - 1]], add=True)
        else:
            def step(i, carry):
                pltpu.sync_copy(rows_a, acc_sh.at[dst_v.at[i]], add=True)
                return carry

            lax.fori_loop(0, nit, step, 0)

        plsc.subcore_barrier()
        pltpu.sync_copy(acc_sh.at[pl.ds(s * rpt, rpt)],
                        out_hbm.at[c, pl.ds(s * rpt, rpt)])

    return pl.kernel(
        body,
        out_type=jax.ShapeDtypeStruct((_NC, npad, width), jnp.float32),
        mesh=mesh,
        scratch_types=[
            pltpu.VMEM((cpt, ch), jnp.int32),
            pltpu.VMEM((cpt, ch), jnp.int32),
            pltpu.VMEM((ch, width), jnp.float32),
            pltpu.VMEM((ch, width), jnp.float32),
            pltpu.VMEM_SHARED((npad, width), jnp.float32),
            pltpu.SemaphoreType.DMA,
            pltpu.SemaphoreType.DMA,
            pltpu.SemaphoreType.DMA,
        ],
        compiler_params=pltpu.CompilerParams(use_tc_tiling_on_sc=False),
    )


_BR = 1000


def _full(shape):
    return pl.BlockSpec(shape, lambda i: (0,) * len(shape))


def _rows(shape3=None, width=128):
    if shape3:
        return pl.BlockSpec((shape3[0], _BR, shape3[2]), lambda i: (0, i, 0))
    return pl.BlockSpec((_BR, width), lambda i: (i, 0))


def _ka_body(x_ref, w_ref, dc_ref, hp_ref, dis_ref):
    deg = 1.0 + dc_ref[0, :, 0:1] + dc_ref[1, :, 0:1]
    dis = lax.rsqrt(deg)
    h = jnp.dot(x_ref[...], w_ref[...], preferred_element_type=jnp.float32)
    hp_ref[...] = h * dis
    dis_ref[...] = dis


def _kb_body(act, pad_out, agg_ref, hp_ref, dis_ref, b_ref, w_ref, out_ref):
    dis = dis_ref[...]
    t = dis * (agg_ref[0] + agg_ref[1] + hp_ref[...]) + b_ref[...]
    t = act(t)
    v = dis * jnp.dot(t, w_ref[...], preferred_element_type=jnp.float32)
    if pad_out:
        v = jnp.concatenate([v, jnp.zeros_like(v)], axis=1)
    out_ref[...] = v


def _kc_body(agg_ref, hp_ref, dis_ref, b_ref, wc_ref, bc_ref, out_ref):
    t = (agg_ref[0] + agg_ref[1] + hp_ref[...])[:, :8]
    t = dis_ref[...] * t + b_ref[...]
    t = jnp.where(t >= 0, t, 0.01 * t)
    o = jnp.dot(t, wc_ref[...], preferred_element_type=jnp.float32) + bc_ref[...]
    out_ref[...] = jnp.where(o > 0, o, jnp.exp(o) - 1.0)


def _act_id(t):
    return t


def _act_tanh(t):
    return jnp.tanh(t)


def _act_leaky(t):
    return jnp.where(t >= 0, t, 0.01 * t)


def kernel(x, edge_index, W1, b1, W2, b2, W3, b3, W4, b4, Wc, bc):
    n, d = x.shape
    e = edge_index.shape[1]
    grid = (n // _BR,)
    npad = -(-n // (8 * _NS)) * (8 * _NS)
    nw = _NC * _NS
    ei = edge_index.astype(jnp.int32)
    ei_w = ei.reshape(2, nw, -1, _CHW)
    ei_n = ei.reshape(2, nw, -1, _CHN)
    ones_blk = jnp.ones((_CHN, 16), jnp.float32)
    zeros16 = jnp.zeros((npad, 16), jnp.float32)
    zerosd = jnp.zeros((npad, d), jnp.float32)

    deg16 = _make_agg(n, e, 16, False, _CHN)
    agg16 = _make_agg(n, e, 16, True, _CHN)
    aggd = _make_agg(n, e, d, True, _CHW)

    dc = deg16(ei_n, ones_blk, zeros16)

    hp1, dis = pl.pallas_call(
        _ka_body,
        grid=grid,
        in_specs=[_rows(width=d), _full((d, d)), _rows((_NC, n, 16))],
        out_specs=[_rows(width=d), _rows(width=1)],
        out_shape=[jax.ShapeDtypeStruct((n, d), jnp.float32),
                   jax.ShapeDtypeStruct((n, 1), jnp.float32)],
    )(x, W1, dc)

    def mid(aggp, hp, b, w, act, dout, pad_out):
        wout = 2 * dout if pad_out else dout
        return pl.pallas_call(
            functools.partial(_kb_body, act, pad_out),
            grid=grid,
            in_specs=[_rows((_NC, n, d)), _rows(width=d), _rows(width=1),
                      _full((1, d)), _full((d, dout))],
            out_specs=_rows(width=wout),
            out_shape=jax.ShapeDtypeStruct((n, wout), jnp.float32),
        )(aggp, hp, dis, b.reshape(1, d), w)

    a1 = aggd(ei_w, hp1, zerosd)
    hp2 = mid(a1, hp1, b1, W2, _act_id, d, False)
    a2 = aggd(ei_w, hp2, zerosd)
    hp3 = mid(a2, hp2, b2, W3, _act_tanh, d, False)
    a3 = aggd(ei_w, hp3, zerosd)
    hp4 = mid(a3, hp3, b3, W4, _act_leaky, 8, True)
    a4 = agg16(ei_n, hp4, zeros16)

    out = pl.pallas_call(
        _kc_body,
        grid=grid,
        in_specs=[_rows((_NC, n, 16)), _rows(width=16), _rows(width=1),
                  _full((1, 8)), _full((8, 1)), _full((1, 1))],
        out_specs=_rows(width=1),
        out_shape=jax.ShapeDtypeStruct((n, 1), jnp.float32),
    )(a4, hp4, dis, b4.reshape(1, 8), Wc, bc.reshape(1, 1))
    return out


# speedup vs baseline: 29.5037x; 1.0456x over previous
import functools

import jax
import jax.numpy as jnp
from jax import lax
from jax.experimental import pallas as pl
from jax.experimental.pallas import tpu as pltpu
from jax.experimental.pallas import tpu_sc as plsc

_NC = 2
_NS = 16
_CHW = 100
_CHN = 400


@functools.lru_cache(maxsize=None)
def _make_agg(n_nodes: int, n_edges: int, width: int, gather: bool, ch: int):
    nw = _NC * _NS
    epw = n_edges // nw
    assert epw * nw == n_edges and epw % ch == 0
    nit = epw // ch
    cpt = nit
    assert nit >= 3
    npad = -(-n_nodes // (8 * _NS)) * (8 * _NS)
    rpt = npad // _NS

    mesh = plsc.VectorSubcoreMesh(
        core_axis_name="c", subcore_axis_name="s",
        num_cores=_NC, num_subcores=_NS)

    def body(ei_hbm, tab_hbm, zeros_hbm, out_hbm,
             src_v, dst_v, rows_a, rows_b, acc_sh, sem_a, sem_b, sem_i):
        c = lax.axis_index("c")
        s = lax.axis_index("s")
        wid = s * _NC + c
        idx = pltpu.async_copy(ei_hbm.at[1, wid], dst_v, sem_i)
        if gather:
            idx2 = pltpu.async_copy(ei_hbm.at[0, wid], src_v, sem_i)
        else:
            idx2 = None
            pltpu.sync_copy(tab_hbm, rows_a)
        pltpu.sync_copy(zeros_hbm.at[pl.ds(s * rpt, rpt)],
                        acc_sh.at[pl.ds(s * rpt, rpt)])
        idx.wait()
        if idx2 is not None:
            idx2.wait()
        plsc.subcore_barrier()

        if gather:
            def g_issue(i, buf, sem):
                pltpu.async_copy(tab_hbm.at[src_v.at[i]], buf, sem)

            def g_wait(i, buf, sem):
                pltpu.make_async_copy(tab_hbm.at[src_v.at[i]], buf, sem).wait()

            g_issue(0, rows_a, sem_a)

            def pair(j, carry):
                i = 2 * j
                g_issue(i + 1, rows_b, sem_b)
                g_wait(i, rows_a, sem_a)
                pltpu.sync_copy(rows_a, acc_sh.at[dst_v.at[i]], add=True)
                g_issue(i + 2, rows_a, sem_a)
                g_wait(i + 1, rows_b, sem_b)
                pltpu.sync_copy(rows_b, acc_sh.at[dst_v.at[i + 1]], add=True)
                return carry

            if nit % 2 == 1:
                lax.fori_loop(0, (nit - 1) // 2, pair, 0)
                g_wait(nit - 1, rows_a, sem_a)
                pltpu.sync_copy(rows_a, acc_sh.at[dst_v.at[nit - 1]], add=True)
            else:
                lax.fori_loop(0, nit // 2 - 1, pair, 0)
                g_issue(nit - 1, rows_b, sem_b)
                g_wait(nit - 2, rows_a, sem_a)
                pltpu.sync_copy(rows_a, acc_sh.at[dst_v.at[nit - 2]], add=True)
                g_wait(nit - 1, rows_b, sem_b)
                pltpu.sync_copy(rows_b, acc_sh.at[dst_v.at[nit - 1]], add=True)
        else:
            def step(i, carry):
                pltpu.sync_copy(rows_a, acc_sh.at[dst_v.at[i]], add=True)
                return carry

            lax.fori_loop(0, nit, step, 0)

        plsc.subcore_barrier()
        pltpu.sync_copy(acc_sh.at[pl.ds(s * rpt, rpt)],
                        out_hbm.at[c, pl.ds(s * rpt, rpt)])

    return pl.kernel(
        body,
        out_type=jax.ShapeDtypeStruct((_NC, npad, width), jnp.float32),
        mesh=mesh,
        scratch_types=[
            pltpu.VMEM((cpt, ch), jnp.int32),
            pltpu.VMEM((cpt, ch), jnp.int32),
            pltpu.VMEM((ch, width), jnp.float32),
            pltpu.VMEM((ch, width), jnp.float32),
            pltpu.VMEM_SHARED((npad, width), jnp.float32),
            pltpu.SemaphoreType.DMA,
            pltpu.SemaphoreType.DMA,
            pltpu.SemaphoreType.DMA,
        ],
        compiler_params=pltpu.CompilerParams(use_tc_tiling_on_sc=False),
    )


_BR = 1000


def _full(shape):
    return pl.BlockSpec(shape, lambda i: (0,) * len(shape))


def _rows(shape3=None, width=128):
    if shape3:
        return pl.BlockSpec((shape3[0], _BR, shape3[2]), lambda i: (0, i, 0))
    return pl.BlockSpec((_BR, width), lambda i: (i, 0))


def _ka_body(x_ref, w_ref, dc_ref, hp_ref, dis_ref):
    deg = 1.0 + dc_ref[0, :, 0:1] + dc_ref[1, :, 0:1]
    dis = lax.rsqrt(deg)
    h = jnp.dot(x_ref[...], w_ref[...], preferred_element_type=jnp.float32)
    hp_ref[...] = h * dis
    dis_ref[...] = dis


def _kb_body(act, pad_out, agg_ref, hp_ref, dis_ref, b_ref, w_ref, out_ref):
    dis = dis_ref[...]
    t = dis * (agg_ref[0] + agg_ref[1] + hp_ref[...]) + b_ref[...]
    t = act(t)
    v = dis * jnp.dot(t, w_ref[...], preferred_element_type=jnp.float32)
    if pad_out:
        v = jnp.concatenate([v, jnp.zeros_like(v)], axis=1)
    out_ref[...] = v


def _kc_body(agg_ref, hp_ref, dis_ref, b_ref, wc_ref, bc_ref, out_ref):
    t = (agg_ref[0] + agg_ref[1] + hp_ref[...])[:, :8]
    t = dis_ref[...] * t + b_ref[...]
    t = jnp.where(t >= 0, t, 0.01 * t)
    o = jnp.dot(t, wc_ref[...], preferred_element_type=jnp.float32) + bc_ref[...]
    out_ref[...] = jnp.where(o > 0, o, jnp.exp(o) - 1.0)


def _act_id(t):
    return t


def _act_tanh(t):
    return jnp.tanh(t)


def _act_leaky(t):
    return jnp.where(t >= 0, t, 0.01 * t)


def kernel(x, edge_index, W1, b1, W2, b2, W3, b3, W4, b4, Wc, bc):
    n, d = x.shape
    e = edge_index.shape[1]
    grid = (n // _BR,)
    npad = -(-n // (8 * _NS)) * (8 * _NS)
    nw = _NC * _NS
    ei = edge_index.astype(jnp.int32)
    ei_w = ei.reshape(2, nw, -1, _CHW)
    ei_n = ei.reshape(2, nw, -1, _CHN)
    ones_blk = jnp.ones((_CHN, 16), jnp.float32)
    zeros16 = jnp.zeros((npad, 16), jnp.float32)
    zerosd = jnp.zeros((npad, d), jnp.float32)

    deg16 = _make_agg(n, e, 16, False, _CHN)
    agg16 = _make_agg(n, e, 16, True, _CHN)
    aggd = _make_agg(n, e, d, True, _CHW)

    dc = deg16(ei_n, ones_blk, zeros16)

    hp1, dis = pl.pallas_call(
        _ka_body,
        grid=grid,
        in_specs=[_rows(width=d), _full((d, d)), _rows((_NC, n, 16))],
        out_specs=[_rows(width=d), _rows(width=1)],
        out_shape=[jax.ShapeDtypeStruct((n, d), jnp.float32),
                   jax.ShapeDtypeStruct((n, 1), jnp.float32)],
    )(x, W1, dc)

    def mid(aggp, hp, b, w, act, dout, pad_out):
        wout = 2 * dout if pad_out else dout
        return pl.pallas_call(
            functools.partial(_kb_body, act, pad_out),
            grid=grid,
            in_specs=[_rows((_NC, n, d)), _rows(width=d), _rows(width=1),
                      _full((1, d)), _full((d, dout))],
            out_specs=_rows(width=wout),
            out_shape=jax.ShapeDtypeStruct((n, wout), jnp.float32),
        )(aggp, hp, dis, b.reshape(1, d), w)

    a1 = aggd(ei_w, hp1, zerosd)
    hp2 = mid(a1, hp1, b1, W2, _act_id, d, False)
    a2 = aggd(ei_w, hp2, zerosd)
    hp3 = mid(a2, hp2, b2, W3, _act_tanh, d, False)
    a3 = aggd(ei_w, hp3, zerosd)
    hp4 = mid(a3, hp3, b3, W4, _act_leaky, 8, True)
    a4 = agg16(ei_n, hp4, zeros16)

    out = pl.pallas_call(
        _kc_body,
        grid=grid,
        in_specs=[_rows((_NC, n, 16)), _rows(width=16), _rows(width=1),
                  _full((1, 8)), _full((8, 1)), _full((1, 1))],
        out_specs=_rows(width=1),
        out_shape=jax.ShapeDtypeStruct((n, 1), jnp.float32),
    )(a4, hp4, dis, b4.reshape(1, 8), Wc, bc.reshape(1, 1))
    return out
